# baseline (speedup 1.0000x reference)
import functools

import jax
import jax.numpy as jnp
from jax import lax
from jax.experimental import pallas as pl
from jax.experimental.pallas import tpu as pltpu
from jax.experimental.pallas import tpu_sc as plsc

C_Z = 128
N_BINS = 64
MIN_D = 0.0
MAX_D = 50.0
N = 512
NTOT = N * N

BIN_W = MAX_D / N_BINS
INV_W = N_BINS / MAX_D
CLIP_HI = MAX_D - 1e-6

NC, NS = 2, 16
NW = NC * NS
ROWS_PER_TILE = NTOT // NW
CHUNK = 128
NCHUNK = ROWS_PER_TILE // CHUNK


def _table_body(we_ref, lnw_ref, lnb_ref, wp_ref, out_ref):
    we = we_ref[...]
    mu = jnp.mean(we, axis=1, keepdims=True)
    var = jnp.mean((we - mu) ** 2, axis=1, keepdims=True)
    x = (we - mu) / jnp.sqrt(var + 1e-5) * lnw_ref[...] + lnb_ref[...]
    out_ref[...] = lax.dot_general(x, wp_ref[...], (((1,), (1,)), ((), ())),
                                   preferred_element_type=jnp.float32)


_table_call = pl.pallas_call(
    _table_body, out_shape=jax.ShapeDtypeStruct((N_BINS, C_Z), jnp.float32))


def _bin16(d):
    d = jnp.minimum(jnp.maximum(d, MIN_D), CLIP_HI)
    c0 = jnp.clip((d * INV_W).astype(jnp.int32), 0, N_BINS - 1)
    e0 = c0.astype(jnp.float32) * BIN_W
    e1 = (c0 + 1).astype(jnp.float32) * BIN_W
    k = jnp.where(d <= e0, c0 - 1, jnp.where(d > e1, c0 + 1, c0))
    return jnp.clip(k, 0, N_BINS - 1)


SLOTS = 4
GROUPS = CHUNK // 16


@functools.cache
def _make_sc_gather():
    scratch = [
        pltpu.VMEM((ROWS_PER_TILE,), jnp.float32),
        pltpu.VMEM((N_BINS * C_Z,), jnp.float32),
    ]
    scratch += [pltpu.VMEM((CHUNK * C_Z,), jnp.float32) for _ in range(SLOTS)]
    scratch += [pltpu.SemaphoreType.DMA for _ in range(SLOTS)]

    @functools.partial(
        pl.kernel,
        mesh=plsc.VectorSubcoreMesh(core_axis_name="c", subcore_axis_name="s"),
        out_type=jax.ShapeDtypeStruct((NTOT * C_Z,), jnp.float32),
        scratch_types=scratch,
        compiler_params=pltpu.CompilerParams(needs_layout_passes=False),
    )
    def _sc_gather(d_hbm, table_hbm, out_hbm, d_v, table_v, *bufs):
        stages = bufs[:SLOTS]
        wsems = bufs[SLOTS:]
        wid = lax.axis_index("s") * NC + lax.axis_index("c")
        base = wid * ROWS_PER_TILE
        pltpu.sync_copy(table_hbm, table_v)
        pltpu.sync_copy(d_hbm.at[pl.ds(base, ROWS_PER_TILE)], d_v)
        lane = lax.iota(jnp.int32, 16)
        _SPLATS = [jnp.full((16,), r, jnp.int32) for r in range(16)]

        def expand_chunk(j, b):
            stage = stages[b]

            @plsc.parallel_loop(0, GROUPS, unroll=1)
            def group_body(g):
                off = j * CHUNK + g * 16
                kvec = _bin16(d_v[pl.ds(off, 16)]) * C_Z
                for r in range(16):
                    base = jnp.take_along_axis(kvec, _SPLATS[r], axis=0,
                                               mode="promise_in_bounds")
                    rowa = base + lane
                    dst = (g * 16 + r) * C_Z
                    for c in range(0, C_Z, 16):
                        vals = plsc.load_gather(table_v, [rowa + c])
                        stage[pl.ds(dst + c, 16)] = vals

        def w_copy(j, b):
            dst = out_hbm.at[pl.ds((base + j * CHUNK) * C_Z, CHUNK * C_Z)]
            return pltpu.make_async_copy(stages[b], dst, wsems[b])

        def chunk_body(t, carry):
            for b in range(SLOTS):
                j = t * SLOTS + b

                @pl.when(t > 0)
                def _():
                    w_copy(j - SLOTS, b).wait()

                expand_chunk(j, b)
                w_copy(j, b).start()
            return carry

        lax.fori_loop(0, NCHUNK // SLOTS, chunk_body, 0)
        for b in range(SLOTS):
            w_copy(NCHUNK - SLOTS + b, b).wait()

    return _sc_gather


def kernel(distance_constraints, W_embed, ln_weight, ln_bias, W_proj):
    table = _table_call(W_embed.T, ln_weight.reshape(1, C_Z),
                        ln_bias.reshape(1, C_Z), W_proj)
    d_flat = distance_constraints.reshape(NTOT)
    out = _make_sc_gather()(d_flat, table.reshape(N_BINS * C_Z))
    return out.reshape(1, N, N, C_Z)


# speedup vs baseline: 1.1103x; 1.1103x over previous
import functools

import jax
import jax.numpy as jnp
from jax import lax
from jax.experimental import pallas as pl
from jax.experimental.pallas import tpu as pltpu
from jax.experimental.pallas import tpu_sc as plsc

C_Z = 128
N_BINS = 64
MIN_D = 0.0
MAX_D = 50.0
N = 512
NTOT = N * N

BIN_W = MAX_D / N_BINS
INV_W = N_BINS / MAX_D
CLIP_HI = MAX_D - 1e-6

NC, NS = 2, 16
NW = NC * NS
ROWS_PER_TILE = NTOT // NW
CHUNK = 256
NCHUNK = ROWS_PER_TILE // CHUNK


def _table_body(we_ref, lnw_ref, lnb_ref, wp_ref, out_ref):
    we = we_ref[...]
    mu = jnp.mean(we, axis=1, keepdims=True)
    var = jnp.mean((we - mu) ** 2, axis=1, keepdims=True)
    x = (we - mu) / jnp.sqrt(var + 1e-5) * lnw_ref[...] + lnb_ref[...]
    out_ref[...] = lax.dot_general(x, wp_ref[...], (((1,), (1,)), ((), ())),
                                   preferred_element_type=jnp.float32)


_table_call = pl.pallas_call(
    _table_body, out_shape=jax.ShapeDtypeStruct((N_BINS, C_Z), jnp.float32))


def _bin16(d):
    d = jnp.minimum(jnp.maximum(d, MIN_D), CLIP_HI)
    c0 = jnp.clip((d * INV_W).astype(jnp.int32), 0, N_BINS - 1)
    e0 = c0.astype(jnp.float32) * BIN_W
    e1 = (c0 + 1).astype(jnp.float32) * BIN_W
    k = jnp.where(d <= e0, c0 - 1, jnp.where(d > e1, c0 + 1, c0))
    return jnp.clip(k, 0, N_BINS - 1)


SUB = 128
NSUB = ROWS_PER_TILE // SUB
TEC_SLOTS = 3
STR_SLOTS = 2
ROUNDS = NSUB // 4
TEC_N = 3 * ROUNDS
SGROUPS = SUB // 16


@functools.cache
def _make_sc_gather():
    scratch = [
        pltpu.VMEM((ROWS_PER_TILE,), jnp.float32),
        pltpu.VMEM((ROWS_PER_TILE,), jnp.int32),
        pltpu.VMEM((N_BINS, C_Z), jnp.float32),
    ]
    scratch += [pltpu.VMEM((SUB, C_Z), jnp.float32) for _ in range(TEC_SLOTS)]
    scratch += [pltpu.VMEM((STR_SLOTS, SUB, C_Z), jnp.float32)]
    scratch += [pltpu.SemaphoreType.DMA for _ in range(TEC_SLOTS)]
    scratch += [pltpu.SemaphoreType.DMA((STR_SLOTS,)) for _ in range(2)]

    @functools.partial(
        pl.kernel,
        mesh=plsc.VectorSubcoreMesh(core_axis_name="c", subcore_axis_name="s"),
        out_type=jax.ShapeDtypeStruct((NTOT, C_Z), jnp.float32),
        scratch_types=scratch,
        compiler_params=pltpu.CompilerParams(needs_layout_passes=False),
    )
    def _sc_gather(d_hbm, table_hbm, out_hbm, d_v, idx_v, table_v, *bufs):
        tstages = bufs[:TEC_SLOTS]
        sstage = bufs[TEC_SLOTS]
        twsems = bufs[TEC_SLOTS + 1:2 * TEC_SLOTS + 1]
        sgsem, swsem = bufs[2 * TEC_SLOTS + 1:]
        wid = lax.axis_index("s") * NC + lax.axis_index("c")
        base = wid * ROWS_PER_TILE
        pltpu.sync_copy(table_hbm, table_v)
        pltpu.sync_copy(d_hbm.at[pl.ds(base, ROWS_PER_TILE)], d_v)
        lane = lax.iota(jnp.int32, 16)
        _SPLATS = [jnp.full((16,), r, jnp.int32) for r in range(16)]

        @plsc.parallel_loop(0, ROWS_PER_TILE // 16, unroll=1)
        def idx_body(i):
            off = i * 16
            idx_v[pl.ds(off, 16)] = _bin16(d_v[pl.ds(off, 16)])

        def expand_sub(cid, b):
            stage = tstages[b]

            @plsc.parallel_loop(0, SGROUPS, unroll=1)
            def group_body(g):
                off = cid * SUB + g * 16
                kvec = idx_v[pl.ds(off, 16)]
                for r in range(16):
                    rowv = jnp.take_along_axis(kvec, _SPLATS[r], axis=0,
                                               mode="promise_in_bounds")
                    dst = g * 16 + r
                    for c in range(0, C_Z, 16):
                        stage[dst, pl.ds(c, 16)] = (
                            plsc.load_gather(table_v, [rowv, lane + c]))

        def t_copy(cid, b):
            dst = out_hbm.at[pl.ds(base + cid * SUB, SUB)]
            return pltpu.make_async_copy(tstages[b], dst, twsems[b])

        def s_gather(sid, b):
            idx_slice = idx_v.at[pl.ds((TEC_N + sid) * SUB, SUB)]
            return pltpu.make_async_copy(table_hbm.at[idx_slice],
                                         sstage.at[b], sgsem.at[b])

        def s_copy(sid, b):
            dst = out_hbm.at[pl.ds(base + (TEC_N + sid) * SUB, SUB)]
            return pltpu.make_async_copy(sstage.at[b], dst, swsem.at[b])

        def round_body(rnd, carry):
            rr = lax.rem(rnd, STR_SLOTS)

            @pl.when(rnd >= STR_SLOTS)
            def _():
                s_copy(rnd - STR_SLOTS, rr).wait()

            s_gather(rnd, rr).start()
            for i in range(3):
                cid = rnd * 3 + i

                @pl.when(rnd > 0)
                def _():
                    t_copy(cid - 3, i).wait()

                expand_sub(cid, i)
                t_copy(cid, i).start()
            s_gather(rnd, rr).wait()
            s_copy(rnd, rr).start()
            return carry

        lax.fori_loop(0, ROUNDS, round_body, 0)
        for i in range(3):
            t_copy(TEC_N - 3 + i, i).wait()
        for rr in range(STR_SLOTS):
            s_copy(ROUNDS - STR_SLOTS + rr, rr).wait()

    return _sc_gather


def kernel(distance_constraints, W_embed, ln_weight, ln_bias, W_proj):
    table = _table_call(W_embed.T, ln_weight.reshape(1, C_Z),
                        ln_bias.reshape(1, C_Z), W_proj)
    d_flat = distance_constraints.reshape(NTOT)
    out = _make_sc_gather()(d_flat, table)
    return out.reshape(1, N, N, C_Z)


# speedup vs baseline: 2.6594x; 2.3951x over previous
import functools

import jax
import jax.numpy as jnp
from jax import lax
from jax.experimental import pallas as pl
from jax.experimental.pallas import tpu as pltpu
from jax.experimental.pallas import tpu_sc as plsc

C_Z = 128
N_BINS = 64
MIN_D = 0.0
MAX_D = 50.0
N = 512
NTOT = N * N

BIN_W = MAX_D / N_BINS
INV_W = N_BINS / MAX_D
CLIP_HI = MAX_D - 1e-6

NC, NS = 2, 16
NW = NC * NS
ROWS_PER_TILE = NTOT // NW
CHUNK = 256
NCHUNK = ROWS_PER_TILE // CHUNK


def _table_body(we_ref, lnw_ref, lnb_ref, wp_ref, out_ref):
    we = we_ref[...]
    mu = jnp.mean(we, axis=1, keepdims=True)
    var = jnp.mean((we - mu) ** 2, axis=1, keepdims=True)
    x = (we - mu) / jnp.sqrt(var + 1e-5) * lnw_ref[...] + lnb_ref[...]
    out_ref[...] = lax.dot_general(x, wp_ref[...], (((1,), (1,)), ((), ())),
                                   preferred_element_type=jnp.float32)


_table_call = pl.pallas_call(
    _table_body, out_shape=jax.ShapeDtypeStruct((N_BINS, C_Z), jnp.float32))


def _bin16(d):
    d = jnp.minimum(jnp.maximum(d, MIN_D), CLIP_HI)
    c0 = jnp.clip((d * INV_W).astype(jnp.int32), 0, N_BINS - 1)
    e0 = c0.astype(jnp.float32) * BIN_W
    e1 = (c0 + 1).astype(jnp.float32) * BIN_W
    k = jnp.where(d <= e0, c0 - 1, jnp.where(d > e1, c0 + 1, c0))
    return jnp.clip(k, 0, N_BINS - 1)


SUB = 128
NSUB = ROWS_PER_TILE // SUB
SLOTS = 4


@functools.cache
def _make_sc_gather():
    scratch = [
        pltpu.VMEM((ROWS_PER_TILE,), jnp.float32),
        pltpu.VMEM((ROWS_PER_TILE,), jnp.int32),
        pltpu.VMEM_SHARED((N_BINS, C_Z), jnp.float32),
        pltpu.VMEM((SLOTS, SUB, C_Z), jnp.float32),
        pltpu.SemaphoreType.DMA((SLOTS,)),
        pltpu.SemaphoreType.DMA((SLOTS,)),
    ]

    @functools.partial(
        pl.kernel,
        mesh=plsc.VectorSubcoreMesh(core_axis_name="c", subcore_axis_name="s"),
        out_type=jax.ShapeDtypeStruct((NTOT, C_Z), jnp.float32),
        scratch_types=scratch,
        compiler_params=pltpu.CompilerParams(needs_layout_passes=False),
    )
    def _sc_gather(d_hbm, table_hbm, out_hbm, d_v, idx_v, table_sh,
                   stage, gsem, wsem):
        sid = lax.axis_index("s")
        wid = sid * NC + lax.axis_index("c")
        base = wid * ROWS_PER_TILE

        @pl.when(sid == 0)
        def _():
            pltpu.sync_copy(table_hbm, table_sh)

        pltpu.sync_copy(d_hbm.at[pl.ds(base, ROWS_PER_TILE)], d_v)

        @plsc.parallel_loop(0, ROWS_PER_TILE // 16, unroll=1)
        def idx_body(i):
            off = i * 16
            idx_v[pl.ds(off, 16)] = _bin16(d_v[pl.ds(off, 16)])

        plsc.subcore_barrier()

        def s_gather(sid_, b):
            idx_slice = idx_v.at[pl.ds(sid_ * SUB, SUB)]
            return pltpu.make_async_copy(table_sh.at[idx_slice],
                                         stage.at[b], gsem.at[b])

        def s_copy(sid_, b):
            dst = out_hbm.at[pl.ds(base + sid_ * SUB, SUB)]
            return pltpu.make_async_copy(stage.at[b], dst, wsem.at[b])

        for b in range(SLOTS):
            s_gather(b, b).start()

        def sub_body(rnd, carry):
            rr = lax.rem(rnd, SLOTS)
            s_gather(rnd, rr).wait()
            s_copy(rnd, rr).start()
            s_copy(rnd, rr).wait()

            @pl.when(rnd + SLOTS < NSUB)
            def _():
                s_gather(rnd + SLOTS, rr).start()

            return carry

        lax.fori_loop(0, NSUB, sub_body, 0)

    return _sc_gather


def kernel(distance_constraints, W_embed, ln_weight, ln_bias, W_proj):
    table = _table_call(W_embed.T, ln_weight.reshape(1, C_Z),
                        ln_bias.reshape(1, C_Z), W_proj)
    d_flat = distance_constraints.reshape(NTOT)
    out = _make_sc_gather()(d_flat, table)
    return out.reshape(1, N, N, C_Z)


# speedup vs baseline: 2.7235x; 1.0241x over previous
import functools

import jax
import jax.numpy as jnp
from jax import lax
from jax.experimental import pallas as pl
from jax.experimental.pallas import tpu as pltpu
from jax.experimental.pallas import tpu_sc as plsc

C_Z = 128
N_BINS = 64
MIN_D = 0.0
MAX_D = 50.0
N = 512
NTOT = N * N

BIN_W = MAX_D / N_BINS
INV_W = N_BINS / MAX_D
CLIP_HI = MAX_D - 1e-6

NC, NS = 2, 16
NW = NC * NS
ROWS_PER_TILE = NTOT // NW
CHUNK = 256
NCHUNK = ROWS_PER_TILE // CHUNK


def _table_body(we_ref, lnw_ref, lnb_ref, wp_ref, out_ref):
    we = we_ref[...]
    mu = jnp.mean(we, axis=1, keepdims=True)
    var = jnp.mean((we - mu) ** 2, axis=1, keepdims=True)
    x = (we - mu) / jnp.sqrt(var + 1e-5) * lnw_ref[...] + lnb_ref[...]
    out_ref[...] = lax.dot_general(x, wp_ref[...], (((1,), (1,)), ((), ())),
                                   preferred_element_type=jnp.float32)


_table_call = pl.pallas_call(
    _table_body, out_shape=jax.ShapeDtypeStruct((N_BINS, C_Z), jnp.float32))


def _bin16(d):
    d = jnp.minimum(jnp.maximum(d, MIN_D), CLIP_HI)
    c0 = jnp.clip((d * INV_W).astype(jnp.int32), 0, N_BINS - 1)
    e0 = c0.astype(jnp.float32) * BIN_W
    e1 = (c0 + 1).astype(jnp.float32) * BIN_W
    k = jnp.where(d <= e0, c0 - 1, jnp.where(d > e1, c0 + 1, c0))
    return jnp.clip(k, 0, N_BINS - 1)


SUB = 128
NSUB = ROWS_PER_TILE // SUB
SLOTS = 6


@functools.cache
def _make_sc_gather():
    scratch = [
        pltpu.VMEM((ROWS_PER_TILE,), jnp.float32),
        pltpu.VMEM((ROWS_PER_TILE,), jnp.int32),
        pltpu.VMEM_SHARED((N_BINS, C_Z), jnp.float32),
        pltpu.VMEM((SLOTS, SUB, C_Z), jnp.float32),
        pltpu.SemaphoreType.DMA((SLOTS,)),
        pltpu.SemaphoreType.DMA((SLOTS,)),
    ]

    @functools.partial(
        pl.kernel,
        mesh=plsc.VectorSubcoreMesh(core_axis_name="c", subcore_axis_name="s"),
        out_type=jax.ShapeDtypeStruct((NTOT, C_Z), jnp.float32),
        scratch_types=scratch,
        compiler_params=pltpu.CompilerParams(needs_layout_passes=False),
    )
    def _sc_gather(d_hbm, table_hbm, out_hbm, d_v, idx_v, table_sh,
                   stage, gsem, wsem):
        sid = lax.axis_index("s")
        wid = sid * NC + lax.axis_index("c")
        base = wid * ROWS_PER_TILE

        @pl.when(sid == 0)
        def _():
            pltpu.sync_copy(table_hbm, table_sh)

        pltpu.sync_copy(d_hbm.at[pl.ds(base, ROWS_PER_TILE)], d_v)

        def compute_idx(sub):
            @plsc.parallel_loop(0, SUB // 16, unroll=1)
            def idx_body(i):
                off = sub * SUB + i * 16
                idx_v[pl.ds(off, 16)] = _bin16(d_v[pl.ds(off, 16)])

        for b in range(SLOTS):
            compute_idx(b)

        plsc.subcore_barrier()

        def s_gather(sid_, b):
            idx_slice = idx_v.at[pl.ds(sid_ * SUB, SUB)]
            return pltpu.make_async_copy(table_sh.at[idx_slice],
                                         stage.at[b], gsem.at[b])

        def s_copy(sid_, b):
            dst = out_hbm.at[pl.ds(base + sid_ * SUB, SUB)]
            return pltpu.make_async_copy(stage.at[b], dst, wsem.at[b])

        for b in range(SLOTS):
            s_gather(b, b).start()

        def sub_body(rnd, carry):
            rr = lax.rem(rnd, SLOTS)
            s_gather(rnd, rr).wait()
            s_copy(rnd, rr).start()

            @pl.when(rnd + SLOTS < NSUB)
            def _():
                compute_idx(rnd + SLOTS)

            s_copy(rnd, rr).wait()

            @pl.when(rnd + SLOTS < NSUB)
            def _():
                s_gather(rnd + SLOTS, rr).start()

            return carry

        lax.fori_loop(0, NSUB, sub_body, 0)

    return _sc_gather


def kernel(distance_constraints, W_embed, ln_weight, ln_bias, W_proj):
    table = _table_call(W_embed.T, ln_weight.reshape(1, C_Z),
                        ln_bias.reshape(1, C_Z), W_proj)
    d_flat = distance_constraints.reshape(NTOT)
    out = _make_sc_gather()(d_flat, table)
    return out.reshape(1, N, N, C_Z)
